# Initial kernel scaffold; baseline (speedup 1.0000x reference)
import jax
import jax.numpy as jnp
from jax.experimental import pallas as pl


def kernel(h, W, b, edge_index):
    raise NotImplementedError("write your pallas kernel here")


# speedup vs baseline: 38.2712x; 38.2712x over previous
import functools

import jax
import jax.numpy as jnp
from jax import lax
from jax.experimental import pallas as pl
from jax.experimental.pallas import tpu as pltpu
from jax.experimental.pallas import tpu_sc as plsc

_N_NODES = 10000
_N_EDGES = 320000
_NC = 2
_NS = 16
_NW = _NC * _NS
_L = 16
_EPW = _N_EDGES // _NW


def _rowsum_body(h_ref, o_ref):
    o_ref[...] = jnp.sum(h_ref[...], axis=1, keepdims=True)


def _rowsum(h):
    out = pl.pallas_call(
        _rowsum_body,
        out_shape=jax.ShapeDtypeStruct((_N_NODES, 1), jnp.float32),
    )(h)
    return out.reshape(_N_NODES)


_gather_mesh = plsc.VectorSubcoreMesh(core_axis_name="c", subcore_axis_name="s")


@functools.partial(
    pl.kernel,
    out_type=jax.ShapeDtypeStruct((_N_EDGES,), jnp.float32),
    mesh=_gather_mesh,
    compiler_params=pltpu.CompilerParams(needs_layout_passes=False),
    scratch_types=[
        pltpu.VMEM((_N_NODES,), jnp.float32),
        pltpu.VMEM((_EPW,), jnp.int32),
        pltpu.VMEM((_EPW,), jnp.float32),
    ],
)
def _gather_kernel(s_hbm, src_hbm, out_hbm, table_v, idx_v, out_v):
    wid = lax.axis_index("s") * _NC + lax.axis_index("c")
    base = wid * _EPW
    pltpu.sync_copy(s_hbm, table_v)
    pltpu.sync_copy(src_hbm.at[pl.ds(base, _EPW)], idx_v)

    def body(g, carry):
        idxs = idx_v[pl.ds(g * _L, _L)]
        out_v[pl.ds(g * _L, _L)] = plsc.load_gather(table_v, [idxs])
        return carry

    lax.fori_loop(0, _EPW // _L, body, 0, unroll=8)
    pltpu.sync_copy(out_v, out_hbm.at[pl.ds(base, _EPW)])


def kernel(h, W, b, edge_index):
    src = edge_index[0].astype(jnp.int32)
    s = _rowsum(h)
    return _gather_kernel(s, src)


# speedup vs baseline: 48.3479x; 1.2633x over previous
import functools

import jax
import jax.numpy as jnp
from jax import lax
from jax.experimental import pallas as pl
from jax.experimental.pallas import tpu as pltpu
from jax.experimental.pallas import tpu_sc as plsc

_N_NODES = 10000
_N_EDGES = 320000
_NC = 2
_NS = 16
_NW = _NC * _NS
_L = 16
_EPW = _N_EDGES // _NW


def _rowsum_body(h_ref, o_ref):
    o_ref[...] = jnp.sum(h_ref[...], axis=1, keepdims=True)


def _rowsum(h):
    out = pl.pallas_call(
        _rowsum_body,
        out_shape=jax.ShapeDtypeStruct((_N_NODES, 1), jnp.float32),
    )(h)
    return out.reshape(_N_NODES)


_gather_mesh = plsc.VectorSubcoreMesh(core_axis_name="c", subcore_axis_name="s")


@functools.partial(
    pl.kernel,
    out_type=jax.ShapeDtypeStruct((_N_EDGES,), jnp.float32),
    mesh=_gather_mesh,
    compiler_params=pltpu.CompilerParams(needs_layout_passes=False),
    scratch_types=[
        pltpu.VMEM((_N_NODES,), jnp.float32),
        pltpu.VMEM((_EPW,), jnp.int32),
        pltpu.VMEM((_EPW,), jnp.float32),
    ],
)
def _gather_kernel(s_hbm, edge_hbm, out_hbm, table_v, idx_v, out_v):
    wid = lax.axis_index("s") * _NC + lax.axis_index("c")
    base = wid * _EPW
    pltpu.sync_copy(s_hbm, table_v)
    pltpu.sync_copy(edge_hbm.at[pl.ds(base, _EPW)], idx_v)

    def body(g, carry):
        idxs = idx_v[pl.ds(g * _L, _L)]
        out_v[pl.ds(g * _L, _L)] = plsc.load_gather(table_v, [idxs])
        return carry

    lax.fori_loop(0, _EPW // _L, body, 0, unroll=8)
    pltpu.sync_copy(out_v, out_hbm.at[pl.ds(base, _EPW)])


def kernel(h, W, b, edge_index):
    s = _rowsum(h)
    return _gather_kernel(s, edge_index.astype(jnp.int32).reshape(-1))


# speedup vs baseline: 58.6099x; 1.2123x over previous
import functools

import jax
import jax.numpy as jnp
from jax import lax
from jax.experimental import pallas as pl
from jax.experimental.pallas import tpu as pltpu
from jax.experimental.pallas import tpu_sc as plsc

_N_NODES = 10000
_N_EDGES = 320000
_NC = 2
_NS = 16
_NW = _NC * _NS
_L = 16

_WBIG = 79 * 128
_WSML = 78 * 128
_NBIG = 4


def _rowsum_body(h_ref, o_ref):
    o_ref[...] = jnp.sum(h_ref[...], axis=1)


def _rowsum(h):
    return pl.pallas_call(
        _rowsum_body,
        out_shape=jax.ShapeDtypeStruct((_N_NODES,), jnp.float32),
    )(h)


_gather_mesh = plsc.VectorSubcoreMesh(core_axis_name="c", subcore_axis_name="s")


@functools.partial(
    pl.kernel,
    out_type=jax.ShapeDtypeStruct((_N_EDGES,), jnp.float32),
    mesh=_gather_mesh,
    compiler_params=pltpu.CompilerParams(needs_layout_passes=False),
    scratch_types=[
        pltpu.VMEM((_N_NODES,), jnp.float32),
        pltpu.VMEM((2, _WBIG), jnp.int32),
        pltpu.VMEM((_WBIG,), jnp.float32),
    ],
)
def _gather_kernel(s_hbm, edge_hbm, out_hbm, table_v, edges_v, out_v):
    wid = lax.axis_index("s") * _NC + lax.axis_index("c")
    big = wid < _NBIG
    base = jnp.where(big, wid * _WBIG, _NBIG * _WBIG + (wid - _NBIG) * _WSML)
    pltpu.sync_copy(s_hbm, table_v)

    def run(width):
        pltpu.sync_copy(edge_hbm.at[:, pl.ds(base, width)],
                        edges_v.at[:, pl.ds(0, width)])

        def body(g, carry):
            idxs = edges_v[0, pl.ds(g * _L, _L)]
            out_v[pl.ds(g * _L, _L)] = plsc.load_gather(table_v, [idxs])
            return carry

        lax.fori_loop(0, width // _L, body, 0, unroll=8)
        pltpu.sync_copy(out_v.at[pl.ds(0, width)],
                        out_hbm.at[pl.ds(base, width)])

    @pl.when(big)
    def _():
        run(_WBIG)

    @pl.when(jnp.logical_not(big))
    def _():
        run(_WSML)


def kernel(h, W, b, edge_index):
    s = _rowsum(h)
    return _gather_kernel(s, edge_index.astype(jnp.int32))


# speedup vs baseline: 69.8611x; 1.1920x over previous
import functools

import jax
import jax.numpy as jnp
from jax import lax
from jax.experimental import pallas as pl
from jax.experimental.pallas import tpu as pltpu
from jax.experimental.pallas import tpu_sc as plsc

_N_NODES = 10000
_N_EDGES = 320000
_NC = 2
_NS = 16
_NW = _NC * _NS
_L = 16

_WBIG = 79 * 128
_WSML = 78 * 128
_NBIG = 4


def _rowsum_body(h_ref, o_ref):
    o_ref[...] = jnp.sum(h_ref[...], axis=1)


_ROWBLK = 2048


def _rowsum(h):
    return pl.pallas_call(
        _rowsum_body,
        grid=(pl.cdiv(_N_NODES, _ROWBLK),),
        in_specs=[pl.BlockSpec((_ROWBLK, 128), lambda i: (i, 0))],
        out_specs=pl.BlockSpec((_ROWBLK,), lambda i: (i,)),
        out_shape=jax.ShapeDtypeStruct((_N_NODES,), jnp.float32),
    )(h)


_gather_mesh = plsc.VectorSubcoreMesh(core_axis_name="c", subcore_axis_name="s")


@functools.partial(
    pl.kernel,
    out_type=jax.ShapeDtypeStruct((_N_EDGES,), jnp.float32),
    mesh=_gather_mesh,
    compiler_params=pltpu.CompilerParams(needs_layout_passes=False),
    scratch_types=[
        pltpu.VMEM((_N_NODES,), jnp.float32),
        pltpu.VMEM((2, _WBIG), jnp.int32),
        pltpu.VMEM((_WBIG,), jnp.float32),
        pltpu.SemaphoreType.DMA,
        pltpu.SemaphoreType.DMA,
    ],
)
def _gather_kernel(s_hbm, edge_hbm, out_hbm, table_v, edges_v, out_v,
                   sem_t, sem_e):
    wid = lax.axis_index("s") * _NC + lax.axis_index("c")
    big = wid < _NBIG
    base = jnp.where(big, wid * _WBIG, _NBIG * _WBIG + (wid - _NBIG) * _WSML)
    tbl_cp = pltpu.async_copy(s_hbm, table_v, sem_t)

    def run(width):
        pltpu.async_copy(edge_hbm.at[:, pl.ds(base, width)],
                         edges_v.at[:, pl.ds(0, width)], sem_e).wait()
        tbl_cp.wait()

        @plsc.parallel_loop(0, width // _L, unroll=8)
        def body(g):
            idxs = edges_v[0, pl.ds(g * _L, _L)]
            out_v[pl.ds(g * _L, _L)] = plsc.load_gather(table_v, [idxs])

        pltpu.sync_copy(out_v.at[pl.ds(0, width)],
                        out_hbm.at[pl.ds(base, width)])

    @pl.when(big)
    def _():
        run(_WBIG)

    @pl.when(jnp.logical_not(big))
    def _():
        run(_WSML)


def kernel(h, W, b, edge_index):
    s = _rowsum(h)
    return _gather_kernel(s, edge_index.astype(jnp.int32))


# speedup vs baseline: 69.9706x; 1.0016x over previous
import functools

import jax
import jax.numpy as jnp
from jax import lax
from jax.experimental import pallas as pl
from jax.experimental.pallas import tpu as pltpu
from jax.experimental.pallas import tpu_sc as plsc

_N_NODES = 10000
_N_EDGES = 320000
_NC = 2
_NS = 16
_NW = _NC * _NS
_L = 16

_WBIG = 79 * 128
_WSML = 78 * 128
_NBIG = 4


def _rowsum_body(h_ref, o_ref):
    ones = jnp.ones((128,), dtype=jnp.float32)
    o_ref[...] = jnp.dot(h_ref[...], ones,
                         preferred_element_type=jnp.float32)


_ROWBLK = 2048


def _rowsum(h):
    return pl.pallas_call(
        _rowsum_body,
        grid=(pl.cdiv(_N_NODES, _ROWBLK),),
        in_specs=[pl.BlockSpec((_ROWBLK, 128), lambda i: (i, 0))],
        out_specs=pl.BlockSpec((_ROWBLK,), lambda i: (i,)),
        out_shape=jax.ShapeDtypeStruct((_N_NODES,), jnp.float32),
    )(h)


_gather_mesh = plsc.VectorSubcoreMesh(core_axis_name="c", subcore_axis_name="s")


@functools.partial(
    pl.kernel,
    out_type=jax.ShapeDtypeStruct((_N_EDGES,), jnp.float32),
    mesh=_gather_mesh,
    compiler_params=pltpu.CompilerParams(needs_layout_passes=False),
    scratch_types=[
        pltpu.VMEM((_N_NODES,), jnp.float32),
        pltpu.VMEM((2, _WBIG), jnp.int32),
        pltpu.VMEM((_WBIG,), jnp.float32),
        pltpu.SemaphoreType.DMA,
        pltpu.SemaphoreType.DMA,
    ],
)
def _gather_kernel(s_hbm, edge_hbm, out_hbm, table_v, edges_v, out_v,
                   sem_t, sem_e):
    wid = lax.axis_index("s") * _NC + lax.axis_index("c")
    big = wid < _NBIG
    base = jnp.where(big, wid * _WBIG, _NBIG * _WBIG + (wid - _NBIG) * _WSML)
    tbl_cp = pltpu.async_copy(s_hbm, table_v, sem_t)

    def run(width):
        pltpu.async_copy(edge_hbm.at[:, pl.ds(base, width)],
                         edges_v.at[:, pl.ds(0, width)], sem_e).wait()
        tbl_cp.wait()

        @plsc.parallel_loop(0, width // _L, unroll=8)
        def body(g):
            idxs = edges_v[0, pl.ds(g * _L, _L)]
            out_v[pl.ds(g * _L, _L)] = plsc.load_gather(table_v, [idxs])

        pltpu.sync_copy(out_v.at[pl.ds(0, width)],
                        out_hbm.at[pl.ds(base, width)])

    @pl.when(big)
    def _():
        run(_WBIG)

    @pl.when(jnp.logical_not(big))
    def _():
        run(_WSML)


def kernel(h, W, b, edge_index):
    s = _rowsum(h)
    return _gather_kernel(s, edge_index.astype(jnp.int32))
